# baseline (speedup 1.0000x reference)
import functools

import jax
import jax.numpy as jnp
from jax.experimental import pallas as pl
from jax.experimental.pallas import tpu as pltpu

N = 10000
BI = 400


def _leaky_relu(x):
    return jnp.where(x >= 0, x, 0.01 * x)


def _softmax(x):
    m = jnp.max(x, axis=1, keepdims=True)
    e = jnp.exp(x - m)
    return e / jnp.sum(e, axis=1, keepdims=True)


def _layer_kernel(x_ref, w_ref, b_ref, adj_ref, out_ref, s_ref, *, activation):
    @pl.when(pl.program_id(0) == 0)
    def _():
        s_ref[...] = jnp.dot(
            x_ref[...], w_ref[...], preferred_element_type=jnp.float32
        )

    acc = jnp.dot(adj_ref[...], s_ref[...], preferred_element_type=jnp.float32)
    out_ref[...] = activation(acc + b_ref[...])


def _gcn_layer(x, w, b, adj, activation):
    n, f_in = x.shape
    f_out = w.shape[1]
    kern = functools.partial(_layer_kernel, activation=activation)
    return pl.pallas_call(
        kern,
        grid=(n // BI,),
        in_specs=[
            pl.BlockSpec((n, f_in), lambda i: (0, 0)),
            pl.BlockSpec((f_in, f_out), lambda i: (0, 0)),
            pl.BlockSpec((1, f_out), lambda i: (0, 0)),
            pl.BlockSpec((BI, n), lambda i: (i, 0)),
        ],
        out_specs=pl.BlockSpec((BI, f_out), lambda i: (i, 0)),
        out_shape=jax.ShapeDtypeStruct((n, f_out), jnp.float32),
        scratch_shapes=[pltpu.VMEM((n, f_out), jnp.float32)],
    )(x, w, b.reshape(1, f_out), adj)


def kernel(y, adj, W1, b1, W2, b2):
    h = _gcn_layer(y, W1, b1, adj, _leaky_relu)
    out = _gcn_layer(h, W2, b2, adj, _softmax)
    return (out, h)
